# Initial kernel scaffold; baseline (speedup 1.0000x reference)
import jax
import jax.numpy as jnp
from jax.experimental import pallas as pl


def kernel(x, edge_index, emb, W1, W2, bn1_g, bn1_b, bn2_g, bn2_b, Wr1, br1, Wr2, br2):
    raise NotImplementedError("write your pallas kernel here")


# speedup vs baseline: 5.8497x; 5.8497x over previous
import functools

import jax
import jax.numpy as jnp
from jax import lax
from jax.experimental import pallas as pl
from jax.experimental.pallas import tpu as pltpu
from jax.experimental.pallas import tpu_sc as plsc

_N = 10000
_E = 320000
_H = 128
_L = 3

_NC = 2
_NS = 16
_NW = _NC * _NS
_EPT = _E // _NW
_K = 80
_NCHUNK = _EPT // _K
_ZR = 16
_NZCH = _N // _ZR


def _sc_agg_body(src_hbm, dst_hbm, h_hbm, out_hbm,
                 src_v, dst_v, rows, zbuf, sem0, acc_sh):
    c = lax.axis_index("c")
    s = lax.axis_index("s")
    wid = s * _NC + c

    pltpu.sync_copy(src_hbm.at[wid], src_v)
    pltpu.sync_copy(dst_hbm.at[wid], dst_v)

    def _zrow(r, carry):
        for j in range(_H // 16):
            zbuf[r, pl.ds(j * 16, 16)] = jnp.zeros((16,), jnp.float32)
        return carry
    lax.fori_loop(0, _ZR, _zrow, 0)

    nj = jnp.where(s < _NZCH % _NS, _NZCH // _NS + 1, _NZCH // _NS)

    def _zchunk(j, carry):
        off = (s + _NS * j) * _ZR
        pltpu.sync_copy(zbuf, acc_sh.at[pl.ds(off, _ZR)])
        return carry
    lax.fori_loop(0, nj, _zchunk, 0)
    plsc.subcore_barrier()

    def _chunk(i, carry):
        pltpu.async_copy(h_hbm.at[src_v.at[i]], rows, sem0)
        pltpu.make_async_copy(h_hbm.at[src_v.at[i]], rows, sem0).wait()
        pltpu.sync_copy(rows, acc_sh.at[dst_v.at[i]], add=True)
        return carry
    lax.fori_loop(0, _NCHUNK, _chunk, 0)

    plsc.subcore_barrier()

    def _wchunk(j, carry):
        off = (s + _NS * j) * _ZR
        pltpu.sync_copy(acc_sh.at[pl.ds(off, _ZR)],
                        out_hbm.at[c].at[pl.ds(off, _ZR)])
        return carry
    lax.fori_loop(0, nj, _wchunk, 0)


@functools.cache
def _make_sc_agg():
    return pl.kernel(
        _sc_agg_body,
        out_type=jax.ShapeDtypeStruct((_NC, _N, _H), jnp.float32),
        mesh=plsc.VectorSubcoreMesh(
            core_axis_name="c", subcore_axis_name="s",
            num_cores=_NC, num_subcores=_NS),
        scratch_types=[
            pltpu.VMEM((_NCHUNK, _K), jnp.int32),
            pltpu.VMEM((_NCHUNK, _K), jnp.int32),
            pltpu.VMEM((_K, _H), jnp.float32),
            pltpu.VMEM((_ZR, _H), jnp.float32),
            pltpu.SemaphoreType.DMA,
            pltpu.VMEM_SHARED((_N, _H), jnp.float32),
        ],
    )


def _sc_agg(src, dst, h):
    return _make_sc_agg()(src, dst, h)


def _emb_body(x_ref, emb_ref, o_ref):
    xv = x_ref[...]
    e0 = emb_ref[0:1, :]
    e1 = emb_ref[1:2, :]
    o_ref[...] = jnp.where(xv == 1, e1, e0)


def _emb_lookup(x2d, emb):
    return pl.pallas_call(
        _emb_body,
        out_shape=jax.ShapeDtypeStruct((_N, _H), jnp.float32),
    )(x2d, emb)


def _layer_body(h_ref, a_ref, w1_ref, w2_ref, g1_ref, b1_ref, g2_ref, b2_ref,
                o_ref):
    z = h_ref[...] + a_ref[0] + a_ref[1]
    z = jnp.dot(z, w1_ref[...], preferred_element_type=jnp.float32)
    mu = jnp.mean(z, axis=0, keepdims=True)
    var = jnp.mean((z - mu) * (z - mu), axis=0, keepdims=True)
    z = (z - mu) / jnp.sqrt(var + 1e-5) * g1_ref[...] + b1_ref[...]
    z = jnp.maximum(z, 0.0)
    z = jnp.dot(z, w2_ref[...], preferred_element_type=jnp.float32)
    mu = jnp.mean(z, axis=0, keepdims=True)
    var = jnp.mean((z - mu) * (z - mu), axis=0, keepdims=True)
    z = (z - mu) / jnp.sqrt(var + 1e-5) * g2_ref[...] + b2_ref[...]
    o_ref[...] = jnp.where(z > 0.0, z, jnp.exp(jnp.minimum(z, 0.0)) - 1.0)


def _layer(h, agg, w1, w2, g1, b1, g2, b2):
    return pl.pallas_call(
        _layer_body,
        out_shape=jax.ShapeDtypeStruct((_N, _H), jnp.float32),
    )(h, agg, w1, w2, g1, b1, g2, b2)


def _readout_body(h0_ref, h1_ref, h2_ref, h3_ref, wr1_ref, br1_ref, wr2_ref,
                  br2_ref, o_ref):
    acc = jnp.dot(h0_ref[...], wr1_ref[0], preferred_element_type=jnp.float32)
    acc += jnp.dot(h1_ref[...], wr1_ref[1], preferred_element_type=jnp.float32)
    acc += jnp.dot(h2_ref[...], wr1_ref[2], preferred_element_type=jnp.float32)
    acc += jnp.dot(h3_ref[...], wr1_ref[3], preferred_element_type=jnp.float32)
    acc = jnp.maximum(acc + br1_ref[...], 0.0)
    o_ref[...] = jnp.dot(acc, wr2_ref[...], preferred_element_type=jnp.float32) + br2_ref[...]


def _readout(h0, h1, h2, h3, wr1, br1, wr2, br2):
    blk = _N // 10
    row_spec = pl.BlockSpec((blk, _H), lambda i: (i, 0))
    return pl.pallas_call(
        _readout_body,
        grid=(10,),
        in_specs=[row_spec, row_spec, row_spec, row_spec,
                  pl.BlockSpec((_L + 1, _H, _H), lambda i: (0, 0, 0)),
                  pl.BlockSpec((1, _H), lambda i: (0, 0)),
                  pl.BlockSpec((_H, 1), lambda i: (0, 0)),
                  pl.BlockSpec((1, 1), lambda i: (0, 0))],
        out_specs=pl.BlockSpec((blk, 1), lambda i: (i, 0)),
        out_shape=jax.ShapeDtypeStruct((_N, 1), jnp.float32),
    )(h0, h1, h2, h3, wr1, br1, wr2, br2)


def kernel(x, edge_index, emb, W1, W2, bn1_g, bn1_b, bn2_g, bn2_b,
           Wr1, br1, Wr2, br2):
    x2d = x.astype(jnp.int32).reshape(_N, 1)
    src = edge_index[0].astype(jnp.int32).reshape(_NW, _NCHUNK, _K)
    dst = edge_index[1].astype(jnp.int32).reshape(_NW, _NCHUNK, _K)

    h = _emb_lookup(x2d, emb)
    hidden = [h]
    for i in range(_L):
        agg = _sc_agg(src, dst, h)
        h = _layer(h, agg, W1[i], W2[i],
                   bn1_g[i:i + 1], bn1_b[i:i + 1],
                   bn2_g[i:i + 1], bn2_b[i:i + 1])
        hidden.append(h)

    wr1 = Wr1.reshape(_L + 1, _H, _H)
    return _readout(hidden[0], hidden[1], hidden[2], hidden[3],
                    wr1, br1.reshape(1, _H), Wr2, br2.reshape(1, 1))
